# Initial kernel scaffold; baseline (speedup 1.0000x reference)
import jax
import jax.numpy as jnp
from jax.experimental import pallas as pl


def kernel(x, edge_index, batch, W1, b1, g1, be1, W2, b2, g2, be2, Wc1, bc1, Wc2, bc2):
    raise NotImplementedError("write your pallas kernel here")


# speedup vs baseline: 16.5300x; 16.5300x over previous
import functools

import numpy as np
import jax
import jax.numpy as jnp
from jax import lax
from jax.experimental import pallas as pl
from jax.experimental.pallas import tpu as pltpu
from jax.experimental.pallas import tpu_sc as plsc

_NC = 2
_NS = 16
_NW = _NC * _NS
_CHUNK = 128
_HP = 128
_G = 64
_BN_SCALE = float(1.0 / np.sqrt(1.0 + 1e-5))
_MM = dict(preferred_element_type=jnp.float32, precision=lax.Precision.HIGHEST)


def _mesh():
    return plsc.VectorSubcoreMesh(core_axis_name="c", subcore_axis_name="s",
                                  num_cores=_NC, num_subcores=_NS)


@functools.lru_cache(maxsize=None)
def _make_deg(np_rows, n_chunks):
    rpt = np_rows // _NS

    @functools.partial(
        pl.kernel,
        out_type=jax.ShapeDtypeStruct((_NC, np_rows), jnp.float32),
        mesh=_mesh(),
        scratch_types=[
            pltpu.VMEM_SHARED((np_rows,), jnp.float32),
            pltpu.VMEM((n_chunks, _CHUNK), jnp.int32),
            pltpu.VMEM((_CHUNK,), jnp.float32),
        ],
    )
    def deg_kernel(dst_hbm, ones_hbm, zeros_hbm, out_hbm, acc_sh, dst_v, ones_v):
        c = lax.axis_index("c")
        s = lax.axis_index("s")
        wid = s * _NC + c
        pltpu.sync_copy(zeros_hbm, acc_sh.at[pl.ds(s * rpt, rpt)])
        pltpu.sync_copy(dst_hbm.at[wid], dst_v)
        pltpu.sync_copy(ones_hbm, ones_v)
        plsc.subcore_barrier()

        def body(j, carry):
            pltpu.sync_copy(ones_v, acc_sh.at[dst_v.at[j]], add=True)
            return carry

        lax.fori_loop(0, n_chunks, body, 0)
        plsc.subcore_barrier()
        pltpu.sync_copy(acc_sh.at[pl.ds(s * rpt, rpt)],
                        out_hbm.at[c, pl.ds(s * rpt, rpt)])

    return deg_kernel


@functools.lru_cache(maxsize=None)
def _make_prop(np_rows, n_chunks):
    rpt = np_rows // _NS

    @functools.partial(
        pl.kernel,
        out_type=jax.ShapeDtypeStruct((_NC, np_rows, _HP), jnp.float32),
        mesh=_mesh(),
        scratch_types=[
            pltpu.VMEM_SHARED((np_rows, _HP), jnp.float32),
            pltpu.VMEM((n_chunks, _CHUNK), jnp.int32),
            pltpu.VMEM((n_chunks, _CHUNK), jnp.int32),
            pltpu.VMEM((_CHUNK, _HP), jnp.float32),
            pltpu.SemaphoreType.DMA,
        ],
    )
    def prop_kernel(table_hbm, src_hbm, dst_hbm, zeros_hbm, out_hbm,
                    acc_sh, src_v, dst_v, rows_v, gsem):
        c = lax.axis_index("c")
        s = lax.axis_index("s")
        wid = s * _NC + c
        pltpu.sync_copy(zeros_hbm, acc_sh.at[pl.ds(s * rpt, rpt)])
        pltpu.sync_copy(src_hbm.at[wid], src_v)
        pltpu.sync_copy(dst_hbm.at[wid], dst_v)
        plsc.subcore_barrier()

        def body(j, carry):
            pltpu.async_copy(table_hbm.at[src_v.at[j]], rows_v, gsem).wait()
            pltpu.sync_copy(rows_v, acc_sh.at[dst_v.at[j]], add=True)
            return carry

        lax.fori_loop(0, n_chunks, body, 0)
        plsc.subcore_barrier()
        pltpu.sync_copy(acc_sh.at[pl.ds(s * rpt, rpt)],
                        out_hbm.at[c, pl.ds(s * rpt, rpt)])

    return prop_kernel


def _mm_scale_body(x_ref, w_ref, deg_ref, hd_ref):
    dinv = 1.0 / jnp.sqrt(1.0 + deg_ref[...])
    hd_ref[...] = jnp.dot(x_ref[...], w_ref[...], **_MM) * dinv


def _layer_body(agg_ref, hd_ref, deg_ref, b_ref, g_ref, be_ref, w_ref, out_ref):
    dinv = 1.0 / jnp.sqrt(1.0 + deg_ref[...])
    t = (agg_ref[0] + agg_ref[1] + hd_ref[...]) * dinv + b_ref[...]
    t = t * (g_ref[...] * _BN_SCALE) + be_ref[...]
    hcur = jnp.maximum(t, 0.0)
    out_ref[...] = jnp.dot(hcur, w_ref[...], **_MM) * dinv


def _final_body(agg_ref, hd_ref, deg_ref, b_ref, g_ref, be_ref, batch_ref,
                wc1_ref, bc1_ref, wc2_ref, bc2_ref, out_ref):
    np_rows = hd_ref.shape[0]
    dinv = 1.0 / jnp.sqrt(1.0 + deg_ref[...])
    t = (agg_ref[0] + agg_ref[1] + hd_ref[...]) * dinv + b_ref[...]
    t = t * (g_ref[...] * _BN_SCALE) + be_ref[...]
    h2 = jnp.maximum(t, 0.0)
    gid = lax.broadcasted_iota(jnp.int32, (_G, np_rows), 0)
    mask = (gid == batch_ref[...]).astype(jnp.float32)
    sums = jnp.dot(mask, h2, **_MM)
    counts = jnp.sum(mask, axis=1, keepdims=True)
    pooled = sums / jnp.maximum(counts, 1.0)
    z = jnp.maximum(jnp.dot(pooled, wc1_ref[...], **_MM) + bc1_ref[...], 0.0)
    out_ref[...] = jnp.dot(z, wc2_ref[...], **_MM) + bc2_ref[...]


def _pad_cols(a, w):
    return jnp.concatenate(
        [a, jnp.zeros(a.shape[:-1] + (w - a.shape[-1],), a.dtype)], axis=-1)


def kernel(x, edge_index, batch, W1, b1, g1, be1, W2, b2, g2, be2,
           Wc1, bc1, Wc2, bc2):
    n, d = x.shape
    e = edge_index.shape[1]
    h1 = W1.shape[1]

    np_rows = (n // 128 + 2) * 128
    ep = -(-e // (_NW * _CHUNK)) * (_NW * _CHUNK)
    pad_e = ep - e
    n_chunks = ep // (_NW * _CHUNK)

    xp = jnp.concatenate([x, jnp.zeros((np_rows - n, d), x.dtype)], axis=0)
    pad_src = jnp.arange(pad_e, dtype=jnp.int32) % n
    pad_dst = n + jnp.arange(pad_e, dtype=jnp.int32) % (np_rows - n)
    src = jnp.concatenate([edge_index[0], pad_src]).reshape(_NW, n_chunks, _CHUNK)
    dst = jnp.concatenate([edge_index[1], pad_dst]).reshape(_NW, n_chunks, _CHUNK)
    batch_p = jnp.concatenate(
        [batch, jnp.full((np_rows - n,), _G, batch.dtype)]).reshape(1, np_rows)

    rpt = np_rows // _NS
    zeros_1d = jnp.zeros((rpt,), jnp.float32)
    ones_chunk = jnp.ones((_CHUNK,), jnp.float32)
    zeros_hp = jnp.zeros((rpt, _HP), jnp.float32)

    W1p = _pad_cols(W1, _HP)
    W2p = _pad_cols(jnp.concatenate(
        [W2, jnp.zeros((_HP - h1, W2.shape[1]), W2.dtype)], axis=0), _HP)
    Wc1p = jnp.concatenate(
        [Wc1, jnp.zeros((_HP - Wc1.shape[0], Wc1.shape[1]), Wc1.dtype)], axis=0)
    b1p = _pad_cols(b1.reshape(1, -1), _HP)
    g1p = _pad_cols(g1.reshape(1, -1), _HP)
    be1p = _pad_cols(be1.reshape(1, -1), _HP)
    b2p = _pad_cols(b2.reshape(1, -1), _HP)
    g2p = _pad_cols(g2.reshape(1, -1), _HP)
    be2p = _pad_cols(be2.reshape(1, -1), _HP)

    degs = _make_deg(np_rows, n_chunks)(dst, ones_chunk, zeros_1d)
    degp = (degs[0] + degs[1]).reshape(np_rows, 1)

    blk = 1024
    grid = (np_rows // blk,)
    hd1 = pl.pallas_call(
        _mm_scale_body,
        grid=grid,
        in_specs=[
            pl.BlockSpec((blk, d), lambda i: (i, 0)),
            pl.BlockSpec((d, _HP), lambda i: (0, 0)),
            pl.BlockSpec((blk, 1), lambda i: (i, 0)),
        ],
        out_specs=pl.BlockSpec((blk, _HP), lambda i: (i, 0)),
        out_shape=jax.ShapeDtypeStruct((np_rows, _HP), jnp.float32),
    )(xp, W1p, degp)

    prop = _make_prop(np_rows, n_chunks)
    agg1 = prop(hd1, src, dst, zeros_hp)

    hd2 = pl.pallas_call(
        _layer_body,
        grid=grid,
        in_specs=[
            pl.BlockSpec((_NC, blk, _HP), lambda i: (0, i, 0)),
            pl.BlockSpec((blk, _HP), lambda i: (i, 0)),
            pl.BlockSpec((blk, 1), lambda i: (i, 0)),
            pl.BlockSpec((1, _HP), lambda i: (0, 0)),
            pl.BlockSpec((1, _HP), lambda i: (0, 0)),
            pl.BlockSpec((1, _HP), lambda i: (0, 0)),
            pl.BlockSpec((_HP, _HP), lambda i: (0, 0)),
        ],
        out_specs=pl.BlockSpec((blk, _HP), lambda i: (i, 0)),
        out_shape=jax.ShapeDtypeStruct((np_rows, _HP), jnp.float32),
    )(agg1, hd1, degp, b1p, g1p, be1p, W2p)

    agg2 = prop(hd2, src, dst, zeros_hp)

    out = pl.pallas_call(
        _final_body,
        out_shape=jax.ShapeDtypeStruct((_G, Wc2.shape[1]), jnp.float32),
    )(agg2, hd2, degp, b2p, g2p, be2p, batch_p, Wc1p,
      bc1.reshape(1, -1), Wc2, bc2.reshape(1, -1))

    return out
